# baseline (speedup 1.0000x reference)
import functools

import jax
import jax.numpy as jnp
from jax import lax
from jax.experimental import pallas as pl
from jax.experimental.pallas import tpu as pltpu
from jax.experimental.pallas import tpu_sc as plsc

M, C, D = 65536, 8, 64
B, S, K = 8, 2048, 4
L = 16
UNROLL = 8
DH = D // 2

NC, NS = 2, 16
NW = NC * NS

BD1, BM1 = 16, 16384


def _mean_body(st_ref, out_ref):
    acc = st_ref[0]
    for c in range(1, C):
        acc = acc + st_ref[c]
    out_ref[...] = acc * (1.0 / C)


def _make_sc_read(d_base):
    def _sc_read(idx_hbm, q_hbm, mn_hbm, out_hbm,
                 mrow_v, idx0, idx1, q0, q1, o0, o1,
                 sem0, sem1, osem0, osem1):
        wid = lax.axis_index("s") * NC + lax.axis_index("c")
        d_glob = d_base + wid
        bufs = ((idx0, q0, o0, sem0, osem0), (idx1, q1, o1, sem1, osem1))

        pltpu.sync_copy(mn_hbm.at[wid], mrow_v)

        def issue(b):
            idx_v, q_v, _, sem, _ = bufs[b % 2]
            pltpu.async_copy(idx_hbm.at[pl.ds(b * K * S, K * S)], idx_v, sem)
            pltpu.async_copy(q_hbm.at[b, d_glob], q_v, sem)

        issue(0)
        for b in range(B):
            idx_v, q_v, out_v, sem, osem = bufs[b % 2]
            if b + 1 < B:
                issue(b + 1)
            if b >= 2:
                pltpu.make_async_copy(out_v, out_hbm.at[b - 2, wid],
                                      osem).wait()
            pltpu.make_async_copy(idx_hbm.at[pl.ds(b * K * S, K * S)], idx_v,
                                  sem).wait()
            pltpu.make_async_copy(q_hbm.at[b, d_glob], q_v, sem).wait()

            @plsc.parallel_loop(0, S, step=L, unroll=UNROLL)
            def _svec(s0, idx_v=idx_v, q_v=q_v, out_v=out_v):
                acc = None
                for k in range(K):
                    iv = idx_v[pl.ds(k * S + s0, L)]
                    g = plsc.load_gather(mrow_v, [iv])
                    acc = g if acc is None else acc + g
                out_v[pl.ds(s0, L)] = acc * q_v[pl.ds(s0, L)]

            pltpu.async_copy(out_v, out_hbm.at[b, wid], osem)

        for b in (B - 2, B - 1):
            _, _, out_v, _, osem = bufs[b % 2]
            pltpu.make_async_copy(out_v, out_hbm.at[b, wid], osem).wait()

    return _sc_read


def _mean_half(states_t, half):
    joff = half * (DH // BD1)
    return pl.pallas_call(
        _mean_body,
        grid=(DH // BD1, M // BM1),
        in_specs=[pl.BlockSpec((C, BD1, BM1),
                               lambda j, i, joff=joff: (0, j + joff, i))],
        out_specs=pl.BlockSpec((BD1, BM1), lambda j, i: (j, i)),
        out_shape=jax.ShapeDtypeStruct((DH, M), jnp.float32),
    )(states_t)


def _sc_half(idx1, q_t, means_h, half):
    f = functools.partial(
        pl.kernel,
        mesh=plsc.VectorSubcoreMesh(core_axis_name="c", subcore_axis_name="s"),
        out_type=jax.ShapeDtypeStruct((B, DH, S), jnp.float32),
        scratch_types=[
            pltpu.VMEM((M,), jnp.float32),
            pltpu.VMEM((K * S,), jnp.int32),
            pltpu.VMEM((K * S,), jnp.int32),
            pltpu.VMEM((S,), jnp.float32),
            pltpu.VMEM((S,), jnp.float32),
            pltpu.VMEM((S,), jnp.float32),
            pltpu.VMEM((S,), jnp.float32),
            pltpu.SemaphoreType.DMA,
            pltpu.SemaphoreType.DMA,
            pltpu.SemaphoreType.DMA,
            pltpu.SemaphoreType.DMA,
        ],
        compiler_params=pltpu.CompilerParams(needs_layout_passes=False),
    )(_make_sc_read(half * DH))
    return f(idx1, q_t, means_h)


@jax.jit
def _run(idx1, q_t, states_t):
    means_a = _mean_half(states_t, 0)
    out_a = _sc_half(idx1, q_t, means_a, 0)
    means_b = _mean_half(states_t, 1)
    out_b = _sc_half(idx1, q_t, means_b, 1)
    return jnp.concatenate([out_a, out_b], axis=1)


def kernel(partition_indices, queries, states):
    states_t = jnp.transpose(states, (1, 2, 0))
    q_t = jnp.transpose(queries, (0, 2, 1))
    idx1 = jnp.transpose(partition_indices, (0, 2, 1)).reshape(B * K * S)
    idx1 = idx1.astype(jnp.int32)
    out_t = _run(idx1, q_t, states_t)
    return jnp.transpose(out_t, (0, 2, 1))


# speedup vs baseline: 1.0289x; 1.0289x over previous
import functools

import jax
import jax.numpy as jnp
from jax import lax
from jax.experimental import pallas as pl
from jax.experimental.pallas import tpu as pltpu
from jax.experimental.pallas import tpu_sc as plsc

M, C, D = 65536, 8, 64
B, S, K = 8, 2048, 4
L = 16
UNROLL = 16

NC, NS = 2, 16
NW = NC * NS
DPW = D // NW

BD1, BM1 = 16, 16384


def _mean_body(st_ref, out_ref):
    acc = st_ref[0]
    for c in range(1, C):
        acc = acc + st_ref[c]
    out_ref[...] = acc * (1.0 / C)


def _sc_read(idx_hbm, q_hbm, mn_hbm, out_hbm,
             mrow_v, idx0, idx1, q0, q1, o0, o1,
             msem, sem0, sem1, osem0, osem1):
    wid = lax.axis_index("s") * NC + lax.axis_index("c")
    bufs = ((idx0, q0, o0, sem0, osem0), (idx1, q1, o1, sem1, osem1))

    for j in range(DPW):
        d = wid * DPW + j

        def issue(b, d=d):
            idx_v, q_v, _, sem, _ = bufs[b % 2]
            pltpu.async_copy(idx_hbm.at[pl.ds(b * K * S, K * S)], idx_v, sem)
            pltpu.async_copy(q_hbm.at[b, d], q_v, sem)

        pltpu.async_copy(mn_hbm.at[d], mrow_v, msem)
        issue(0)
        pltpu.make_async_copy(mn_hbm.at[d], mrow_v, msem).wait()

        for b in range(B):
            idx_v, q_v, out_v, sem, osem = bufs[b % 2]
            if b + 1 < B:
                issue(b + 1)
            if b >= 2:
                pltpu.make_async_copy(out_v, out_hbm.at[b - 2, d], osem).wait()
            pltpu.make_async_copy(idx_hbm.at[pl.ds(b * K * S, K * S)], idx_v,
                                  sem).wait()
            pltpu.make_async_copy(q_hbm.at[b, d], q_v, sem).wait()

            @plsc.parallel_loop(0, S, step=L, unroll=UNROLL)
            def _svec(s0, idx_v=idx_v, q_v=q_v, out_v=out_v):
                acc = None
                for k in range(K):
                    iv = idx_v[pl.ds(k * S + s0, L)]
                    g = plsc.load_gather(mrow_v, [iv])
                    acc = g if acc is None else acc + g
                out_v[pl.ds(s0, L)] = acc * q_v[pl.ds(s0, L)]

            pltpu.async_copy(out_v, out_hbm.at[b, d], osem)

        for b in (B - 2, B - 1):
            _, _, out_v, _, osem = bufs[b % 2]
            pltpu.make_async_copy(out_v, out_hbm.at[b, d], osem).wait()


@jax.jit
def _run(idx1, q_t, states_t):
    means_t = pl.pallas_call(
        _mean_body,
        grid=(D // BD1, M // BM1),
        in_specs=[pl.BlockSpec((C, BD1, BM1), lambda j, i: (0, j, i))],
        out_specs=pl.BlockSpec((BD1, BM1), lambda j, i: (j, i)),
        out_shape=jax.ShapeDtypeStruct((D, M), jnp.float32),
    )(states_t)

    f = functools.partial(
        pl.kernel,
        mesh=plsc.VectorSubcoreMesh(core_axis_name="c", subcore_axis_name="s"),
        out_type=jax.ShapeDtypeStruct((B, D, S), jnp.float32),
        scratch_types=[
            pltpu.VMEM((M,), jnp.float32),
            pltpu.VMEM((K * S,), jnp.int32),
            pltpu.VMEM((K * S,), jnp.int32),
            pltpu.VMEM((S,), jnp.float32),
            pltpu.VMEM((S,), jnp.float32),
            pltpu.VMEM((S,), jnp.float32),
            pltpu.VMEM((S,), jnp.float32),
            pltpu.SemaphoreType.DMA,
            pltpu.SemaphoreType.DMA,
            pltpu.SemaphoreType.DMA,
            pltpu.SemaphoreType.DMA,
            pltpu.SemaphoreType.DMA,
        ],
        compiler_params=pltpu.CompilerParams(needs_layout_passes=False),
    )(_sc_read)
    return f(idx1, q_t, means_t)


def kernel(partition_indices, queries, states):
    states_t = jnp.transpose(states, (1, 2, 0))
    q_t = jnp.transpose(queries, (0, 2, 1))
    idx1 = jnp.transpose(partition_indices, (0, 2, 1)).reshape(B * K * S)
    idx1 = idx1.astype(jnp.int32)
    out_t = _run(idx1, q_t, states_t)
    return jnp.transpose(out_t, (0, 2, 1))


# speedup vs baseline: 1.0596x; 1.0298x over previous
import functools

import jax
import jax.numpy as jnp
from jax import lax
from jax.experimental import pallas as pl
from jax.experimental.pallas import tpu as pltpu
from jax.experimental.pallas import tpu_sc as plsc

M, C, D = 65536, 8, 64
B, S, K = 8, 2048, 4
L = 16
UNROLL = 8

NC, NS = 2, 16
NW = NC * NS
DPW = D // NW

BD1, BM1 = 16, 16384


def _mean_body(st_ref, out_ref):
    acc = st_ref[0]
    for c in range(1, C):
        acc = acc + st_ref[c]
    out_ref[...] = acc * (1.0 / C)


def _sc_read(idx_hbm, q_hbm, mn_hbm, out_hbm,
             mrow_v, idx0, idx1, q0, q1, o0, o1,
             msem, sem0, sem1, osem0, osem1):
    wid = lax.axis_index("s") * NC + lax.axis_index("c")
    bufs = ((idx0, q0, o0, sem0, osem0), (idx1, q1, o1, sem1, osem1))

    for j in range(DPW):
        d = wid * DPW + j

        def issue(b, d=d):
            idx_v, q_v, _, sem, _ = bufs[b % 2]
            pltpu.async_copy(idx_hbm.at[pl.ds(b * K * S, K * S)], idx_v, sem)
            pltpu.async_copy(q_hbm.at[b, d], q_v, sem)

        pltpu.async_copy(mn_hbm.at[d], mrow_v, msem)
        issue(0)
        pltpu.make_async_copy(mn_hbm.at[d], mrow_v, msem).wait()

        for b in range(B):
            idx_v, q_v, out_v, sem, osem = bufs[b % 2]
            if b + 1 < B:
                issue(b + 1)
            if b >= 2:
                pltpu.make_async_copy(out_v, out_hbm.at[b - 2, d], osem).wait()
            pltpu.make_async_copy(idx_hbm.at[pl.ds(b * K * S, K * S)], idx_v,
                                  sem).wait()
            pltpu.make_async_copy(q_hbm.at[b, d], q_v, sem).wait()

            @plsc.parallel_loop(0, S, step=L, unroll=UNROLL)
            def _svec(s0, idx_v=idx_v, q_v=q_v, out_v=out_v):
                acc = None
                for k in range(K):
                    iv = idx_v[pl.ds(k * S + s0, L)]
                    g = plsc.load_gather(mrow_v, [iv])
                    acc = g if acc is None else acc + g
                out_v[pl.ds(s0, L)] = acc * q_v[pl.ds(s0, L)]

            pltpu.async_copy(out_v, out_hbm.at[b, d], osem)

        for b in (B - 2, B - 1):
            _, _, out_v, _, osem = bufs[b % 2]
            pltpu.make_async_copy(out_v, out_hbm.at[b, d], osem).wait()


@jax.jit
def _run(idx1, q_t, states_t):
    means_t = pl.pallas_call(
        _mean_body,
        grid=(D // BD1, M // BM1),
        in_specs=[pl.BlockSpec((C, BD1, BM1), lambda j, i: (0, j, i))],
        out_specs=pl.BlockSpec((BD1, BM1), lambda j, i: (j, i)),
        out_shape=jax.ShapeDtypeStruct((D, M), jnp.float32),
    )(states_t)

    f = functools.partial(
        pl.kernel,
        mesh=plsc.VectorSubcoreMesh(core_axis_name="c", subcore_axis_name="s"),
        out_type=jax.ShapeDtypeStruct((B, D, S), jnp.float32),
        scratch_types=[
            pltpu.VMEM((M,), jnp.float32),
            pltpu.VMEM((K * S,), jnp.int32),
            pltpu.VMEM((K * S,), jnp.int32),
            pltpu.VMEM((S,), jnp.float32),
            pltpu.VMEM((S,), jnp.float32),
            pltpu.VMEM((S,), jnp.float32),
            pltpu.VMEM((S,), jnp.float32),
            pltpu.SemaphoreType.DMA,
            pltpu.SemaphoreType.DMA,
            pltpu.SemaphoreType.DMA,
            pltpu.SemaphoreType.DMA,
            pltpu.SemaphoreType.DMA,
        ],
        compiler_params=pltpu.CompilerParams(needs_layout_passes=False),
    )(_sc_read)
    return f(idx1, q_t, means_t)


def kernel(partition_indices, queries, states):
    states_t = jnp.transpose(states, (1, 2, 0))
    q_t = jnp.transpose(queries, (0, 2, 1))
    idx1 = jnp.transpose(partition_indices, (0, 2, 1)).reshape(B * K * S)
    idx1 = idx1.astype(jnp.int32)
    out_t = _run(idx1, q_t, states_t)
    return jnp.transpose(out_t, (0, 2, 1))
